# Initial kernel scaffold; baseline (speedup 1.0000x reference)
import functools

import jax
import jax.numpy as jnp
from jax import lax
from jax.experimental import pallas as pl
from jax.experimental.pallas import tpu as pltpu
from jax.experimental.pallas import tpu_sc as plsc

N_ATOMS = 100000
N_TYPES = 100
N_PROPS = 128
N_SYSTEMS = 1024

_NC = 2
_NS = 16
_NW = _NC * _NS

_CHUNK = 3136
_LAST = N_ATOMS - (_NW - 1) * _CHUNK
_HIST = N_SYSTEMS * N_TYPES


def _sc_histogram(atom_types, system_ids, type_to_index):
    mesh = plsc.VectorSubcoreMesh(core_axis_name="c", subcore_axis_name="s")

    @functools.partial(
        pl.kernel,
        mesh=mesh,
        out_type=jax.ShapeDtypeStruct((_NW, _HIST), jnp.float32),
        scratch_types=[
            pltpu.VMEM((_CHUNK,), jnp.int32),
            pltpu.VMEM((_CHUNK,), jnp.int32),
            pltpu.VMEM((N_TYPES,), jnp.int32),
            pltpu.VMEM((_HIST,), jnp.float32),
        ],
    )
    def hist_kernel(types_hbm, sys_hbm, tti_hbm, out_hbm,
                    types_v, sys_v, tti_v, hist_v):
        wid = lax.axis_index("s") * _NC + lax.axis_index("c")
        base = wid * _CHUNK
        is_last = wid == _NW - 1

        @pl.when(jnp.logical_not(is_last))
        def _():
            pltpu.sync_copy(types_hbm.at[pl.ds(base, _CHUNK)], types_v)
            pltpu.sync_copy(sys_hbm.at[pl.ds(base, _CHUNK)], sys_v)

        @pl.when(is_last)
        def _():
            pltpu.sync_copy(types_hbm.at[pl.ds(base, _LAST)],
                            types_v.at[pl.ds(0, _LAST)])
            pltpu.sync_copy(sys_hbm.at[pl.ds(base, _LAST)],
                            sys_v.at[pl.ds(0, _LAST)])

        pltpu.sync_copy(tti_hbm, tti_v)

        zeros = jnp.zeros((16,), jnp.float32)

        def zero_body(i, carry):
            b = i * 64
            for j in range(4):
                hist_v[pl.ds(b + j * 16, 16)] = zeros
            return carry

        lax.fori_loop(0, _HIST // 64, zero_body, 0)

        ones = jnp.ones((16,), jnp.float32)
        n_vecs = jnp.where(is_last, _LAST // 16, _CHUNK // 16)

        def atom_body(i, carry):
            o = i * 16
            t = types_v[pl.ds(o, 16)]
            s = sys_v[pl.ds(o, 16)]
            idx = plsc.load_gather(tti_v, [t])
            flat = s * N_TYPES + idx
            plsc.addupdate_scatter(hist_v, [flat], ones)
            return carry

        lax.fori_loop(0, n_vecs, atom_body, 0)

        pltpu.sync_copy(hist_v, out_hbm.at[wid])

    return hist_kernel(atom_types, system_ids, type_to_index)


_BS = 256


def _tc_body(counts_ref, w_ref, o_ref):
    acc = jnp.sum(counts_ref[...], axis=0)
    o_ref[...] = jnp.dot(acc, w_ref[...], preferred_element_type=jnp.float32)


def _tc_reduce_matmul(counts, weights):
    return pl.pallas_call(
        _tc_body,
        grid=(N_SYSTEMS // _BS,),
        in_specs=[
            pl.BlockSpec((_NW, _BS, N_TYPES), lambda i: (0, i, 0)),
            pl.BlockSpec((N_TYPES, N_PROPS), lambda i: (0, 0)),
        ],
        out_specs=pl.BlockSpec((_BS, N_PROPS), lambda i: (i, 0)),
        out_shape=jax.ShapeDtypeStruct((N_SYSTEMS, N_PROPS), jnp.float32),
    )(counts, weights)


def kernel(atom_types, system_ids, type_to_index, weights):
    counts = _sc_histogram(atom_types.astype(jnp.int32),
                           system_ids.astype(jnp.int32),
                           type_to_index.astype(jnp.int32))
    counts = counts.reshape(_NW, N_SYSTEMS, N_TYPES)
    return _tc_reduce_matmul(counts, weights.astype(jnp.float32))


# speedup vs baseline: 15.4279x; 15.4279x over previous
import functools

import jax
import jax.numpy as jnp
from jax import lax
from jax.experimental import pallas as pl
from jax.experimental.pallas import tpu as pltpu
from jax.experimental.pallas import tpu_sc as plsc

N_ATOMS = 100000
N_TYPES = 100
N_PROPS = 128
N_SYSTEMS = 1024

_NC = 2
_NS = 16
_NW = _NC * _NS

_CHUNK = 3136
_LAST = N_ATOMS - (_NW - 1) * _CHUNK
_HIST = N_SYSTEMS * N_TYPES


def _sc_histogram(atom_types, system_ids):
    mesh = plsc.VectorSubcoreMesh(core_axis_name="c", subcore_axis_name="s")

    @functools.partial(
        pl.kernel,
        mesh=mesh,
        out_type=jax.ShapeDtypeStruct((_NW, _HIST), jnp.float32),
        scratch_types=[
            pltpu.VMEM((_CHUNK,), jnp.int32),
            pltpu.VMEM((_CHUNK,), jnp.int32),
            pltpu.VMEM((_HIST,), jnp.float32),
        ],
        compiler_params=pltpu.CompilerParams(needs_layout_passes=False),
    )
    def hist_kernel(types_hbm, sys_hbm, out_hbm, types_v, sys_v, hist_v):
        wid = lax.axis_index("s") * _NC + lax.axis_index("c")
        base = wid * _CHUNK
        is_last = wid == _NW - 1

        @pl.when(jnp.logical_not(is_last))
        def _():
            pltpu.sync_copy(types_hbm.at[pl.ds(base, _CHUNK)], types_v)
            pltpu.sync_copy(sys_hbm.at[pl.ds(base, _CHUNK)], sys_v)

        @pl.when(is_last)
        def _():
            pltpu.sync_copy(types_hbm.at[pl.ds(base, _LAST)],
                            types_v.at[pl.ds(0, _LAST)])
            pltpu.sync_copy(sys_hbm.at[pl.ds(base, _LAST)],
                            sys_v.at[pl.ds(0, _LAST)])

        zeros = jnp.zeros((16,), jnp.float32)

        def zero_body(i, carry):
            b = i * 64
            for j in range(4):
                hist_v[pl.ds(b + j * 16, 16)] = zeros
            return carry

        lax.fori_loop(0, _HIST // 64, zero_body, 0)

        ones = jnp.ones((16,), jnp.float32)
        n_vecs = jnp.where(is_last, _LAST // 16, _CHUNK // 16)

        def atom_body(i, carry):
            o = i * 16
            t = types_v[pl.ds(o, 16)]
            s = sys_v[pl.ds(o, 16)]
            flat = s * N_TYPES + t
            plsc.addupdate_scatter(hist_v, [flat], ones)
            return carry

        lax.fori_loop(0, n_vecs, atom_body, 0)

        pltpu.sync_copy(hist_v, out_hbm.at[wid])

    return hist_kernel(atom_types, system_ids)


_BS = 256


def _tc_body(counts_ref, tti_ref, w_ref, o_ref):
    acc = jnp.sum(counts_ref[...], axis=0)
    tti = tti_ref[...]
    onehot = (tti == lax.broadcasted_iota(jnp.int32, (N_TYPES, N_TYPES), 1)
              ).astype(jnp.float32)
    w_eff = jnp.dot(onehot, w_ref[...], preferred_element_type=jnp.float32,
                    precision=lax.Precision.HIGHEST)
    o_ref[...] = jnp.dot(acc, w_eff, preferred_element_type=jnp.float32,
                         precision=lax.Precision.HIGHEST)


def _tc_reduce_matmul(counts, tti, weights):
    return pl.pallas_call(
        _tc_body,
        grid=(N_SYSTEMS // _BS,),
        in_specs=[
            pl.BlockSpec((_NW, _BS, N_TYPES), lambda i: (0, i, 0)),
            pl.BlockSpec((N_TYPES, 1), lambda i: (0, 0)),
            pl.BlockSpec((N_TYPES, N_PROPS), lambda i: (0, 0)),
        ],
        out_specs=pl.BlockSpec((_BS, N_PROPS), lambda i: (i, 0)),
        out_shape=jax.ShapeDtypeStruct((N_SYSTEMS, N_PROPS), jnp.float32),
    )(counts, tti, weights)


def kernel(atom_types, system_ids, type_to_index, weights):
    counts = _sc_histogram(atom_types.astype(jnp.int32),
                           system_ids.astype(jnp.int32))
    counts = counts.reshape(_NW, N_SYSTEMS, N_TYPES)
    tti = type_to_index.astype(jnp.int32).reshape(N_TYPES, 1)
    return _tc_reduce_matmul(counts, tti, weights.astype(jnp.float32))


# speedup vs baseline: 26.2990x; 1.7046x over previous
import functools

import jax
import jax.numpy as jnp
from jax import lax
from jax.experimental import pallas as pl
from jax.experimental.pallas import tpu as pltpu
from jax.experimental.pallas import tpu_sc as plsc

N_ATOMS = 100000
N_TYPES = 100
N_PROPS = 128
N_SYSTEMS = 1024

_NC = 2
_NS = 16
_NW = _NC * _NS

_CHUNK = 3136
_LAST = N_ATOMS - (_NW - 1) * _CHUNK
_TPAD = 128
_HALF = N_SYSTEMS // 2


def _sc_histogram(atom_types, system_ids):
    mesh = plsc.VectorSubcoreMesh(core_axis_name="c", subcore_axis_name="s")

    @functools.partial(
        pl.kernel,
        mesh=mesh,
        out_type=jax.ShapeDtypeStruct((_NW, N_SYSTEMS, _TPAD), jnp.float32),
        scratch_types=[
            pltpu.VMEM((_CHUNK,), jnp.int32),
            pltpu.VMEM((_CHUNK,), jnp.int32),
            pltpu.VMEM((_HALF, _TPAD), jnp.float32),
        ],
        compiler_params=pltpu.CompilerParams(needs_layout_passes=False),
    )
    def hist_kernel(types_hbm, sys_hbm, out_hbm, types_v, sys_v, hist_v):
        wid = lax.axis_index("s") * _NC + lax.axis_index("c")
        base = wid * _CHUNK
        is_last = wid == _NW - 1

        @pl.when(jnp.logical_not(is_last))
        def _():
            pltpu.sync_copy(types_hbm.at[pl.ds(base, _CHUNK)], types_v)
            pltpu.sync_copy(sys_hbm.at[pl.ds(base, _CHUNK)], sys_v)

        @pl.when(is_last)
        def _():
            pltpu.sync_copy(types_hbm.at[pl.ds(base, _LAST)],
                            types_v.at[pl.ds(0, _LAST)])
            pltpu.sync_copy(sys_hbm.at[pl.ds(base, _LAST)],
                            sys_v.at[pl.ds(0, _LAST)])

        zeros = jnp.zeros((16,), jnp.float32)
        ones = jnp.ones((16,), jnp.float32)
        n_vecs = jnp.where(is_last, _LAST // 16, _CHUNK // 16)

        for p in range(2):
            lo = p * _HALF

            def zero_body(i, carry):
                for j in range(_TPAD // 16):
                    hist_v[i, pl.ds(j * 16, 16)] = zeros
                return carry

            lax.fori_loop(0, _HALF, zero_body, 0)

            def atom_body(i, carry):
                o = i * 16
                t = types_v[pl.ds(o, 16)]
                s = sys_v[pl.ds(o, 16)] - lo
                mask = jnp.logical_and(s >= 0, s < _HALF)
                plsc.addupdate_scatter(hist_v, [s, t], ones, mask=mask)
                return carry

            lax.fori_loop(0, n_vecs, atom_body, 0)

            pltpu.sync_copy(hist_v, out_hbm.at[wid, pl.ds(lo, _HALF), :])

    return hist_kernel(atom_types, system_ids)


_BS = 128


def _tc_body(counts_ref, tti_ref, w_ref, o_ref):
    acc = jnp.sum(counts_ref[...], axis=0)
    tti = tti_ref[...]
    onehot = (tti == lax.broadcasted_iota(jnp.int32, (_TPAD, _TPAD), 1)
              ).astype(jnp.float32)
    w_eff = jnp.dot(onehot, w_ref[...], preferred_element_type=jnp.float32,
                    precision=lax.Precision.HIGHEST)
    o_ref[...] = jnp.dot(acc, w_eff, preferred_element_type=jnp.float32,
                         precision=lax.Precision.HIGHEST)


def _tc_reduce_matmul(counts, tti_pad, w_pad):
    return pl.pallas_call(
        _tc_body,
        grid=(N_SYSTEMS // _BS,),
        in_specs=[
            pl.BlockSpec((_NW, _BS, _TPAD), lambda i: (0, i, 0)),
            pl.BlockSpec((_TPAD, 1), lambda i: (0, 0)),
            pl.BlockSpec((_TPAD, N_PROPS), lambda i: (0, 0)),
        ],
        out_specs=pl.BlockSpec((_BS, N_PROPS), lambda i: (i, 0)),
        out_shape=jax.ShapeDtypeStruct((N_SYSTEMS, N_PROPS), jnp.float32),
    )(counts, tti_pad, w_pad)


def kernel(atom_types, system_ids, type_to_index, weights):
    counts = _sc_histogram(atom_types.astype(jnp.int32),
                           system_ids.astype(jnp.int32))
    tti_pad = jnp.concatenate(
        [type_to_index.astype(jnp.int32),
         jnp.full((_TPAD - N_TYPES,), -1, jnp.int32)]).reshape(_TPAD, 1)
    w_pad = jnp.zeros((_TPAD, N_PROPS), jnp.float32).at[:N_TYPES].set(
        weights.astype(jnp.float32))
    return _tc_reduce_matmul(counts, tti_pad, w_pad)


# speedup vs baseline: 28.8399x; 1.0966x over previous
import functools

import jax
import jax.numpy as jnp
from jax import lax
from jax.experimental import pallas as pl
from jax.experimental.pallas import tpu as pltpu
from jax.experimental.pallas import tpu_sc as plsc

N_ATOMS = 100000
N_TYPES = 100
N_PROPS = 128
N_SYSTEMS = 1024

_NC = 2
_NS = 16
_NW = _NC * _NS

_S = 4
_A = _NW // _S
_R = N_SYSTEMS // _S
_TPAD = 128

_CHUNK = 12512
_LAST = N_ATOMS - (_A - 1) * _CHUNK


def _sc_histogram(atom_types, system_ids):
    mesh = plsc.VectorSubcoreMesh(core_axis_name="c", subcore_axis_name="s")

    @functools.partial(
        pl.kernel,
        mesh=mesh,
        out_type=jax.ShapeDtypeStruct((_NW, _R, _TPAD), jnp.float32),
        scratch_types=[
            pltpu.VMEM((_CHUNK,), jnp.int32),
            pltpu.VMEM((_CHUNK + 16,), jnp.int32),
            pltpu.VMEM((_R, _TPAD), jnp.float32),
        ],
        compiler_params=pltpu.CompilerParams(needs_layout_passes=False),
    )
    def hist_kernel(types_hbm, sys_hbm, out_hbm, types_v, sys_v, hist_v):
        wid = lax.axis_index("s") * _NC + lax.axis_index("c")
        shard = wid % _A
        lo = (wid // _A) * _R
        base = shard * _CHUNK
        is_last = shard == _A - 1
        n = jnp.where(is_last, _LAST, _CHUNK)

        @pl.when(jnp.logical_not(is_last))
        def _():
            pltpu.sync_copy(types_hbm.at[pl.ds(base, _CHUNK)], types_v)
            pltpu.sync_copy(sys_hbm.at[pl.ds(base, _CHUNK)],
                            sys_v.at[pl.ds(0, _CHUNK)])

        @pl.when(is_last)
        def _():
            pltpu.sync_copy(types_hbm.at[pl.ds(base, _LAST)],
                            types_v.at[pl.ds(0, _LAST)])
            pltpu.sync_copy(sys_hbm.at[pl.ds(base, _LAST)],
                            sys_v.at[pl.ds(0, _LAST)])

        zeros = jnp.zeros((16,), jnp.float32)
        ones = jnp.ones((16,), jnp.float32)

        def zero_body(i, carry):
            for j in range(_TPAD // 16):
                hist_v[i, pl.ds(j * 16, 16)] = zeros
            return carry

        lax.fori_loop(0, _R, zero_body, 0)

        def lower_bound(bound):
            def body(_, ab):
                a, b = ab
                m = (a + b) // 2
                go_right = sys_v[pl.ds(m, 16)][0] < bound
                return (jnp.where(go_right, m + 1, a),
                        jnp.where(go_right, b, m))
            a, _b = lax.fori_loop(0, 15, body, (0, n))
            return a

        i_lo = lower_bound(lo)
        i_hi = lower_bound(lo + _R)

        def atom_body(i, carry):
            o = i * 16
            t = types_v[pl.ds(o, 16)]
            s = sys_v[pl.ds(o, 16)] - lo
            mask = jnp.logical_and(s >= 0, s < _R)
            plsc.addupdate_scatter(hist_v, [s, t], ones, mask=mask)
            return carry

        lax.fori_loop(i_lo // 16, (i_hi + 15) // 16, atom_body, 0)

        pltpu.sync_copy(hist_v, out_hbm.at[wid])

    return hist_kernel(atom_types, system_ids)


_BS = 128
_BPG = _R // _BS


def _tc_body(counts_ref, tti_ref, w_ref, o_ref):
    acc = jnp.sum(counts_ref[...], axis=0)
    tti = tti_ref[...]
    onehot = (tti == lax.broadcasted_iota(jnp.int32, (_TPAD, _TPAD), 1)
              ).astype(jnp.float32)
    w_eff = jnp.dot(onehot, w_ref[...], preferred_element_type=jnp.float32,
                    precision=lax.Precision.HIGHEST)
    o_ref[...] = jnp.dot(acc, w_eff, preferred_element_type=jnp.float32,
                         precision=lax.Precision.HIGHEST)


def _tc_reduce_matmul(counts, tti_pad, w_pad):
    return pl.pallas_call(
        _tc_body,
        grid=(N_SYSTEMS // _BS,),
        in_specs=[
            pl.BlockSpec((_A, _BS, _TPAD), lambda i: (i // _BPG, i % _BPG, 0)),
            pl.BlockSpec((_TPAD, 1), lambda i: (0, 0)),
            pl.BlockSpec((_TPAD, N_PROPS), lambda i: (0, 0)),
        ],
        out_specs=pl.BlockSpec((_BS, N_PROPS), lambda i: (i, 0)),
        out_shape=jax.ShapeDtypeStruct((N_SYSTEMS, N_PROPS), jnp.float32),
    )(counts, tti_pad, w_pad)


def kernel(atom_types, system_ids, type_to_index, weights):
    counts = _sc_histogram(atom_types.astype(jnp.int32),
                           system_ids.astype(jnp.int32))
    tti_pad = jnp.concatenate(
        [type_to_index.astype(jnp.int32),
         jnp.full((_TPAD - N_TYPES,), -1, jnp.int32)]).reshape(_TPAD, 1)
    w_pad = jnp.zeros((_TPAD, N_PROPS), jnp.float32).at[:N_TYPES].set(
        weights.astype(jnp.float32))
    return _tc_reduce_matmul(counts, tti_pad, w_pad)
